# baseline (speedup 1.0000x reference)
import functools

import jax
import jax.numpy as jnp
from jax import lax
from jax.experimental import pallas as pl
from jax.experimental.pallas import tpu as pltpu
from jax.experimental.pallas import tpu_sc as plsc

D_MODEL = 64
NUM_WORKERS = 32
CHUNK = 128
NBUF = 4
TBUF = 2

_PARAMS = pltpu.CompilerParams(
    use_tc_tiling_on_sc=True, needs_layout_passes=False)


@functools.cache
def _build_format(vocab: int):
    n_blocks = vocab // CHUNK
    n_tail = vocab - n_blocks * CHUNK
    n_iters = 2 * ((n_blocks + 2 * NUM_WORKERS - 1) // (2 * NUM_WORKERS))
    mesh = plsc.VectorSubcoreMesh(core_axis_name="c", subcore_axis_name="s")

    @functools.partial(
        pl.kernel,
        mesh=mesh,
        out_type=jax.ShapeDtypeStruct((vocab, 2 * D_MODEL), jnp.float32),
        scratch_types=[
            pltpu.VMEM((2, D_MODEL, CHUNK), jnp.float32),
            pltpu.VMEM((2, CHUNK, CHUNK + 1), jnp.float32),
            pltpu.VMEM((D_MODEL, n_tail), jnp.float32),
        ] + [pltpu.SemaphoreType.DMA] * 4,
        compiler_params=_PARAMS,
    )
    def format_kernel(wt_hbm, wp_hbm, slab_v, buf_v, tail_v, *sems):
        rsem = sems[:2]
        wsem = sems[2:]
        wid = lax.axis_index("s") * 2 + lax.axis_index("c")
        lane = lax.iota(jnp.int32, 16)
        rowvecs = [lane + g * 16 for g in range(CHUNK // 16)]

        def v0_of(i):
            b = jnp.minimum(i * NUM_WORKERS + wid, n_blocks - 1)
            return pl.multiple_of(b * CHUNK, CHUNK)

        def fire_read(i, p):
            pltpu.async_copy(
                wt_hbm.at[:, pl.ds(v0_of(i), CHUNK)], slab_v.at[p], rsem[p])

        for p in range(2):
            fire_read(p, p)

        def body(it, carry):
            for p in range(2):
                i = it * 2 + p
                v0 = v0_of(i)

                pltpu.make_async_copy(
                    wt_hbm.at[:, pl.ds(v0, CHUNK)], slab_v.at[p], rsem[p]
                ).wait()

                @pl.when(it >= 1)
                def _():
                    pltpu.make_async_copy(
                        buf_v.at[p, :, pl.ds(0, CHUNK)],
                        wp_hbm.at[pl.ds(v0_of(i - 2), CHUNK), :],
                        wsem[p],
                    ).wait()

                @plsc.parallel_loop(0, D_MODEL, unroll=8)
                def _tr(c):
                    chi = c + D_MODEL
                    for g in range(CHUNK // 16):
                        vals = slab_v[p, c, pl.ds(g * 16, 16)]
                        plsc.store_scatter(
                            buf_v.at[p], [rowvecs[g], lane * 0 + c], vals)
                        plsc.store_scatter(
                            buf_v.at[p], [rowvecs[g], lane * 0 + chi], vals)

                pltpu.async_copy(
                    buf_v.at[p, :, pl.ds(0, CHUNK)],
                    wp_hbm.at[pl.ds(v0, CHUNK), :],
                    wsem[p],
                )

                @pl.when(i + 2 < n_iters)
                def _():
                    fire_read(i + 2, p)

            return carry

        lax.fori_loop(0, n_iters // 2, body, 0)

        for p in range(2):
            i = n_iters - 2 + p
            pltpu.make_async_copy(
                buf_v.at[p, :, pl.ds(0, CHUNK)],
                wp_hbm.at[pl.ds(v0_of(i), CHUNK), :],
                wsem[p],
            ).wait()

        @pl.when(wid == 0)
        def _():
            tail0 = n_blocks * CHUNK
            pltpu.sync_copy(wt_hbm.at[:, pl.ds(tail0, n_tail)], tail_v)

            @plsc.parallel_loop(0, D_MODEL, unroll=8)
            def _trt(c):
                chi = c + D_MODEL
                for g in range(n_tail // 16):
                    vals = tail_v[c, pl.ds(g * 16, 16)]
                    plsc.store_scatter(
                        buf_v.at[0], [rowvecs[g], lane * 0 + c], vals)
                    plsc.store_scatter(
                        buf_v.at[0], [rowvecs[g], lane * 0 + chi], vals)

            pltpu.sync_copy(
                buf_v.at[0, pl.ds(0, n_tail), pl.ds(0, CHUNK)],
                wp_hbm.at[pl.ds(tail0, n_tail), :],
            )

    return format_kernel


@functools.cache
def _build_gather(n_b: int, n_s: int):
    b_per_w = n_b // NUM_WORKERS
    k_per_s = b_per_w // CHUNK
    idx_per_w = b_per_w * n_s
    assert k_per_s == NBUF
    mesh = plsc.VectorSubcoreMesh(core_axis_name="c", subcore_axis_name="s")

    @functools.partial(
        pl.kernel,
        mesh=mesh,
        out_type=jax.ShapeDtypeStruct((n_s, D_MODEL, n_b), jnp.float32),
        scratch_types=[
            pltpu.VMEM((idx_per_w,), jnp.int32),
            pltpu.VMEM((NBUF, CHUNK, 2 * D_MODEL), jnp.float32),
            pltpu.VMEM((TBUF, D_MODEL, CHUNK + 1), jnp.float32),
        ] + [pltpu.SemaphoreType.DMA] * (NBUF + TBUF),
        compiler_params=_PARAMS,
    )
    def gather_kernel(idx_hbm, table_hbm, out_hbm, idx_v, rows_v, tp_v, *sems):
        gsem = sems[:NBUF]
        osem = sems[NBUF:]
        wid = lax.axis_index("s") * 2 + lax.axis_index("c")
        base_b = wid * b_per_w
        pltpu.sync_copy(idx_hbm.at[wid], idx_v)

        lane = lax.iota(jnp.int32, 16)
        rvecs = [lane + k * 16 for k in range(D_MODEL // 16)]

        def fire_gather(j, p):
            pltpu.async_copy(
                table_hbm.at[idx_v.at[pl.ds(j * CHUNK, CHUNK)]],
                rows_v.at[p],
                gsem[p],
            )

        for p in range(NBUF):
            fire_gather(p, p)

        def body(g, carry):
            for p in range(NBUF):
                j = g * NBUF + p
                tb = p % TBUF
                col0 = pl.multiple_of(base_b + p * CHUNK, CHUNK)

                pltpu.make_async_copy(
                    table_hbm.at[idx_v.at[pl.ds(j * CHUNK, CHUNK)]],
                    rows_v.at[p],
                    gsem[p],
                ).wait()

                @pl.when(j >= TBUF)
                def _():
                    jp = j - TBUF
                    pltpu.make_async_copy(
                        tp_v.at[tb, :, pl.ds(0, CHUNK)],
                        out_hbm.at[jp // k_per_s, :,
                                   pl.ds(pl.multiple_of(
                                       base_b + (jp % k_per_s) * CHUNK,
                                       CHUNK), CHUNK)],
                        osem[tb],
                    ).wait()

                @plsc.parallel_loop(0, CHUNK, unroll=8)
                def _tr(r):
                    rsp = lane * 0 + r
                    for k in range(D_MODEL // 16):
                        vals = rows_v[p, r, pl.ds(k * 16, 16)]
                        plsc.store_scatter(tp_v.at[tb], [rvecs[k], rsp], vals)

                pltpu.async_copy(
                    tp_v.at[tb, :, pl.ds(0, CHUNK)],
                    out_hbm.at[g, :, pl.ds(col0, CHUNK)],
                    osem[tb],
                )

                @pl.when(g + 1 < n_s)
                def _():
                    fire_gather(j + NBUF, p)

            return carry

        lax.fori_loop(0, n_s, body, 0)

        for t in range(TBUF):
            j = n_s * NBUF - TBUF + t
            pltpu.make_async_copy(
                tp_v.at[j % TBUF, :, pl.ds(0, CHUNK)],
                out_hbm.at[j // k_per_s, :,
                           pl.ds(pl.multiple_of(
                               base_b + (j % k_per_s) * CHUNK, CHUNK),
                               CHUNK)],
                osem[j % TBUF],
            ).wait()

    return gather_kernel


def kernel(token_ids, weights):
    n_b, n_s = token_ids.shape
    b_per_w = n_b // NUM_WORKERS
    vocab = weights.shape[0]
    wp = _build_format(vocab)(weights.T)
    idx = token_ids.T.reshape(n_s, NUM_WORKERS, b_per_w)
    idx = idx.transpose(1, 0, 2).reshape(NUM_WORKERS, n_s * b_per_w)
    idx = idx.astype(jnp.int32)
    out = _build_gather(n_b, n_s)(idx, wp)
    return out.transpose(2, 0, 1)


# speedup vs baseline: 2.6765x; 2.6765x over previous
import functools

import jax
import jax.numpy as jnp
from jax import lax
from jax.experimental import pallas as pl
from jax.experimental.pallas import tpu as pltpu
from jax.experimental.pallas import tpu_sc as plsc

D_MODEL = 64
NUM_WORKERS = 32
CHUNK = 128
NBUF = 4
TBUF = 2


@functools.cache
def _build(n_b: int, n_s: int):
    b_per_w = n_b // NUM_WORKERS
    k_per_s = b_per_w // CHUNK
    idx_per_w = b_per_w * n_s
    assert k_per_s == NBUF
    mesh = plsc.VectorSubcoreMesh(core_axis_name="c", subcore_axis_name="s")

    @functools.partial(
        pl.kernel,
        mesh=mesh,
        out_type=jax.ShapeDtypeStruct((n_s, D_MODEL, n_b), jnp.float32),
        scratch_types=[
            pltpu.VMEM((idx_per_w,), jnp.int32),
            pltpu.VMEM((NBUF, CHUNK, D_MODEL), jnp.float32),
            pltpu.VMEM((TBUF, D_MODEL, CHUNK + 1), jnp.float32),
        ] + [pltpu.SemaphoreType.DMA] * (NBUF + TBUF),
        compiler_params=pltpu.CompilerParams(
            use_tc_tiling_on_sc=False, needs_layout_passes=False),
    )
    def gather_kernel(idx_hbm, table_hbm, out_hbm, idx_v, rows_v, tp_v, *sems):
        gsem = sems[:NBUF]
        osem = sems[NBUF:]
        wid = lax.axis_index("s") * 2 + lax.axis_index("c")
        base_b = wid * b_per_w
        pltpu.sync_copy(idx_hbm.at[wid], idx_v)

        lane = lax.iota(jnp.int32, 16)
        rvecs = [lane + bg * 16 for bg in range(CHUNK // 16)]

        def fire_gather(j, p):
            pltpu.async_copy(
                table_hbm.at[idx_v.at[pl.ds(j * CHUNK, CHUNK)]],
                rows_v.at[p],
                gsem[p],
            )

        for p in range(NBUF):
            fire_gather(p, p)

        def body(g, carry):
            for p in range(NBUF):
                j = g * NBUF + p
                tb = p % TBUF
                col0 = base_b + p * CHUNK

                pltpu.make_async_copy(
                    table_hbm.at[idx_v.at[pl.ds(j * CHUNK, CHUNK)]],
                    rows_v.at[p],
                    gsem[p],
                ).wait()

                @pl.when(j >= TBUF)
                def _():
                    jp = j - TBUF
                    pltpu.make_async_copy(
                        tp_v.at[tb, :, pl.ds(0, CHUNK)],
                        out_hbm.at[jp // k_per_s, :,
                                   pl.ds(base_b + (jp % k_per_s) * CHUNK,
                                         CHUNK)],
                        osem[tb],
                    ).wait()

                @plsc.parallel_loop(0, CHUNK, unroll=8)
                def _tr(r):
                    rsp = lane * 0 + r
                    for k in range(D_MODEL // 16):
                        vals = rows_v[p, r, pl.ds(k * 16, 16)]
                        plsc.store_scatter(tp_v.at[tb], [rvecs[k], rsp], vals)

                pltpu.async_copy(
                    tp_v.at[tb, :, pl.ds(0, CHUNK)],
                    out_hbm.at[g, :, pl.ds(col0, CHUNK)],
                    osem[tb],
                )

                @pl.when(g + 1 < n_s)
                def _():
                    fire_gather(j + NBUF, p)

            return carry

        lax.fori_loop(0, n_s, body, 0)

        for t in range(TBUF):
            j = n_s * NBUF - TBUF + t
            pltpu.make_async_copy(
                tp_v.at[j % TBUF, :, pl.ds(0, CHUNK)],
                out_hbm.at[j // k_per_s, :,
                           pl.ds(base_b + (j % k_per_s) * CHUNK, CHUNK)],
                osem[j % TBUF],
            ).wait()

    return gather_kernel


def kernel(token_ids, weights):
    n_b, n_s = token_ids.shape
    b_per_w = n_b // NUM_WORKERS
    idx = token_ids.T.reshape(n_s, NUM_WORKERS, b_per_w)
    idx = idx.transpose(1, 0, 2).reshape(NUM_WORKERS, n_s * b_per_w)
    idx = idx.astype(jnp.int32)
    out = _build(n_b, n_s)(idx, weights)
    return out.transpose(2, 0, 1)
